# baseline (device time: 22094 ns/iter reference)
import jax
import jax.numpy as jnp
from jax import lax
from jax.experimental import pallas as pl
from jax.experimental.pallas import tpu as pltpu

N_DEV = 16
NZ, NQ = 4, 4


def _gelu(y):
    c = 0.7978845608028654
    return 0.5 * y * (1.0 + jnp.tanh(c * (y + 0.044715 * y * y * y)))


def kernel(x, w_mat):
    m_per, k = x.shape
    _, n_per = w_mat.shape
    m = N_DEV * m_per

    def body(x_hbm, w_hbm, out_hbm, x_vmem, w_vmem, w_bf, y_vmem, xg_ref,
             copy_sems, z_send_sems, dir_send_sems, fwd_send_sems,
             recv_sems, dummy_sem):
        my = lax.axis_index("i")
        my_z = my // NQ
        my_q = my % NQ
        right = my_z * NQ + (my_q + 1) % NQ
        left = my_z * NQ + (my_q + 3) % NQ
        left_col = (my_q + 3) % NQ
        right_col = (my_q + 1) % NQ

        def z_near(j):
            if j == 0:
                return jnp.where(my_z == 0, 1, my_z - 1)
            if j == 1:
                return jnp.where(my_z <= 1, 2, jnp.where(my_z == 2, 3, 1))
            return jnp.where(my_z <= 1, 3, 0)

        cp_x = pltpu.make_async_copy(x_hbm, x_vmem, copy_sems.at[0])
        cp_x.start()
        cp_w = pltpu.make_async_copy(w_hbm, w_vmem, copy_sems.at[1])
        cp_w.start()

        barrier_sem = pltpu.get_barrier_semaphore()
        for j in range(NZ - 1):
            zp = z_near(j)
            pl.semaphore_signal(
                barrier_sem, inc=1,
                device_id=(zp * NQ + my_q,),
                device_id_type=pl.DeviceIdType.MESH,
            )
        for nbr in (left, right):
            pl.semaphore_signal(
                barrier_sem, inc=1,
                device_id=(nbr,), device_id_type=pl.DeviceIdType.MESH,
            )
        pl.semaphore_wait(barrier_sem, 5)

        cp_x.wait()
        xg_ref[pl.ds(my * m_per, m_per), :] = x_vmem[...].astype(jnp.bfloat16)
        cp_w.wait()
        w_bf[...] = w_vmem[...].astype(jnp.bfloat16)

        def compute_block(o):
            y_vmem[pl.ds(o * m_per, m_per), :] = _gelu(
                lax.dot_general(
                    xg_ref[pl.ds(o * m_per, m_per), :], w_bf[...],
                    (((1,), (0,)), ((), ())),
                    preferred_element_type=jnp.float32,
                )
            )

        compute_block(my)

        sends = []

        def rdma(o, sem, target):
            r = pltpu.make_async_remote_copy(
                src_ref=xg_ref.at[pl.ds(o * m_per, m_per), :],
                dst_ref=xg_ref.at[pl.ds(o * m_per, m_per), :],
                send_sem=sem,
                recv_sem=recv_sems.at[o],
                device_id=(target,),
                device_id_type=pl.DeviceIdType.MESH,
            )
            r.start()
            sends.append(r)

        def wait_origin(o):
            r = pltpu.make_async_remote_copy(
                src_ref=xg_ref.at[pl.ds(o * m_per, m_per), :],
                dst_ref=xg_ref.at[pl.ds(o * m_per, m_per), :],
                send_sem=dummy_sem,
                recv_sem=recv_sems.at[o],
                device_id=(my,),
                device_id_type=pl.DeviceIdType.MESH,
            )
            r.wait_recv()

        for j in (2, 1, 0):
            zp = z_near(j)
            rdma(my, z_send_sems.at[zp], zp * NQ + my_q)

        rdma(my, dir_send_sems.at[my_z, 0], right)
        rdma(my, dir_send_sems.at[my_z, 1], left)

        for j in range(NZ - 1):
            zp = z_near(j)
            o = zp * NQ + my_q
            wait_origin(o)
            rdma(o, dir_send_sems.at[zp, 0], right)
            rdma(o, dir_send_sems.at[zp, 1], left)
            compute_block(o)

        for zf in (0, 1):
            o = zf * NQ + left_col
            wait_origin(o)
            rdma(o, fwd_send_sems.at[zf], right)
            compute_block(o)
        for zf in (2, 3):
            o = zf * NQ + right_col
            wait_origin(o)
            rdma(o, fwd_send_sems.at[zf], left)
            compute_block(o)

        for zf in (2, 3):
            o = zf * NQ + left_col
            wait_origin(o)
            compute_block(o)
        for zf in (0, 1):
            o = zf * NQ + right_col
            wait_origin(o)
            compute_block(o)
        diag_col = (my_q + 2) % NQ
        for jj in range(NZ):
            zd = my_z if jj == 0 else z_near(jj - 1)
            o = zd * NQ + diag_col
            wait_origin(o)
            compute_block(o)

        for r in sends:
            r.wait_send()

        cp_out = pltpu.make_async_copy(y_vmem, out_hbm, copy_sems.at[2])
        cp_out.start()
        cp_out.wait()

    return pl.pallas_call(
        body,
        out_shape=jax.ShapeDtypeStruct((m, n_per), jnp.float32),
        in_specs=[
            pl.BlockSpec(memory_space=pl.ANY),
            pl.BlockSpec(memory_space=pl.ANY),
        ],
        out_specs=pl.BlockSpec(memory_space=pl.ANY),
        scratch_shapes=[
            pltpu.VMEM((m_per, k), jnp.float32),
            pltpu.VMEM((k, n_per), jnp.float32),
            pltpu.VMEM((k, n_per), jnp.bfloat16),
            pltpu.VMEM((m, n_per), jnp.float32),
            pltpu.VMEM((m, k), jnp.bfloat16),
            pltpu.SemaphoreType.DMA((3,)),
            pltpu.SemaphoreType.DMA((NZ,)),
            pltpu.SemaphoreType.DMA((NZ, 2)),
            pltpu.SemaphoreType.DMA((NZ,)),
            pltpu.SemaphoreType.DMA((N_DEV,)),
            pltpu.SemaphoreType.DMA,
        ],
        compiler_params=pltpu.CompilerParams(collective_id=0),
    )(x, w_mat)
